# baseline (speedup 1.0000x reference)
import functools

import jax
import jax.numpy as jnp
from jax import lax
from jax.experimental import pallas as pl
from jax.experimental.pallas import tpu as pltpu
from jax.experimental.pallas import tpu_sc as plsc

M, C, F = 4, 128, 50000
K = 3
NC, NS, L = 2, 16, 16
NW = NC * NS
ROWS_PER_W = (M * C) // NW
W_PER_MESH = C // ROWS_PER_W

FP = 50048
NG = FP // (2 * L)
NCHUNK = 4
GC = NG // NCHUNK
FC = GC * 2 * L
F_LAST = F - (NCHUNK - 1) * FC


def _sc_body(fea_hbm, ring_hbm, out_hbm, fea_buf, ia0, ia1, ia2, ib0, ib1,
             ib2, out_a, out_b, sem_ia, sem_ib, sem_oa, sem_ob):
    cid = lax.axis_index("c")
    sid = lax.axis_index("s")
    wid = cid * NS + sid
    m = wid // W_PER_MESH
    c0 = (wid % W_PER_MESH) * ROWS_PER_W

    ibufs = [((ia0, ia1, ia2), sem_ia), ((ib0, ib1, ib2), sem_ib)]
    obufs = [(out_a, sem_oa), (out_b, sem_ob)]

    iota2 = lax.iota(jnp.int32, L) * 2

    def idx_dma(fc):
        ibs, s_i = ibufs[fc % 2]
        return [
            pltpu.async_copy(
                ring_hbm.at[m, k, pl.ds(fc * GC * L, GC * L)], ibs[k], s_i)
            for k in range(K)
        ]

    def row_body(r, carry):
        row = m * C + c0 + r
        pltpu.sync_copy(fea_hbm.at[pl.ds(row * F, F)], fea_buf.at[pl.ds(0, F)])

        h_idx = [None] * NCHUNK
        h_out = [None] * NCHUNK
        h_idx[0] = idx_dma(0)

        for fc in range(NCHUNK):
            ibs = ibufs[fc % 2][0]
            ob, s_o = obufs[fc % 2]
            if fc + 1 < NCHUNK:
                h_idx[fc + 1] = idx_dma(fc + 1)
            for h in h_idx[fc]:
                h.wait()
            if fc >= 2:
                h_out[fc - 2].wait()
            cb = fc * FC

            @plsc.parallel_loop(0, GC, 1, unroll=8)
            def group_body(g, ibs=ibs, ob=ob, cb=cb):
                x0 = ibs[0][pl.ds(g * L, L)]
                x1 = ibs[1][pl.ds(g * L, L)]
                x2 = ibs[2][pl.ds(g * L, L)]
                lo0 = x0 & 0xFFFF
                lo1 = x1 & 0xFFFF
                lo2 = x2 & 0xFFFF
                hi0 = lax.shift_right_logical(x0, 16)
                hi1 = lax.shift_right_logical(x1, 16)
                hi2 = lax.shift_right_logical(x2, 16)
                ev = iota2 + (cb + g * 2 * L)
                od = ev + 1
                v_ev = plsc.load_gather(fea_buf, [ev])
                v_od = plsc.load_gather(fea_buf, [od])
                g0 = plsc.load_gather(fea_buf, [lo0])
                g1 = plsc.load_gather(fea_buf, [lo1])
                g2 = plsc.load_gather(fea_buf, [lo2])
                g3 = plsc.load_gather(fea_buf, [hi0])
                g4 = plsc.load_gather(fea_buf, [hi1])
                g5 = plsc.load_gather(fea_buf, [hi2])
                oev = iota2 + g * 2 * L
                plsc.store_scatter(ob, [oev], jnp.maximum(
                    jnp.maximum(v_ev, g0), jnp.maximum(g1, g2)))
                plsc.store_scatter(ob, [oev + 1], jnp.maximum(
                    jnp.maximum(v_od, g3), jnp.maximum(g4, g5)))

            n_out = FC if fc + 1 < NCHUNK else F_LAST
            h_out[fc] = pltpu.async_copy(
                ob.at[pl.ds(0, n_out)],
                out_hbm.at[pl.ds(row * F + cb, n_out)], s_o)

        h_out[NCHUNK - 2].wait()
        h_out[NCHUNK - 1].wait()
        return carry

    lax.fori_loop(0, ROWS_PER_W, row_body, 0)


_sc_pool = functools.partial(
    pl.kernel,
    mesh=plsc.VectorSubcoreMesh(core_axis_name="c", subcore_axis_name="s"),
    compiler_params=pltpu.CompilerParams(
        needs_layout_passes=False, use_tc_tiling_on_sc=False),
    out_type=jax.ShapeDtypeStruct((M * C * F,), jnp.float32),
    scratch_types=[
        pltpu.VMEM((FP,), jnp.float32),
        pltpu.VMEM((GC * L,), jnp.int32),
        pltpu.VMEM((GC * L,), jnp.int32),
        pltpu.VMEM((GC * L,), jnp.int32),
        pltpu.VMEM((GC * L,), jnp.int32),
        pltpu.VMEM((GC * L,), jnp.int32),
        pltpu.VMEM((GC * L,), jnp.int32),
        pltpu.VMEM((FC,), jnp.float32),
        pltpu.VMEM((FC,), jnp.float32),
        pltpu.SemaphoreType.DMA,
        pltpu.SemaphoreType.DMA,
        pltpu.SemaphoreType.DMA,
        pltpu.SemaphoreType.DMA,
    ],
)(_sc_body)


def kernel(fea, ring_n):
    ring_p = jnp.pad(ring_n, ((0, 0), (0, FP - F), (0, 0)))
    packed = ring_p[:, 0::2, :] | (ring_p[:, 1::2, :] << 16)
    packed = jnp.transpose(packed, (0, 2, 1))
    return _sc_pool(fea.reshape(-1), packed).reshape(M, C, F)


# speedup vs baseline: 1.0375x; 1.0375x over previous
import functools

import jax
import jax.numpy as jnp
from jax import lax
from jax.experimental import pallas as pl
from jax.experimental.pallas import tpu as pltpu
from jax.experimental.pallas import tpu_sc as plsc

M, C, F = 4, 128, 50000
K = 3
NC, NS, L = 2, 16, 16
NW = NC * NS
ROWS_PER_W = (M * C) // NW
W_PER_MESH = C // ROWS_PER_W

FP = 50048
NG = FP // (2 * L)
NCHUNK = 4
GC = NG // NCHUNK
FC = GC * 2 * L
F_LAST = F - (NCHUNK - 1) * FC


def _sc_body(fea_hbm, ring_hbm, out_hbm, fea_buf, ia0, ia1, ia2, ib0, ib1,
             ib2, out_a, out_b, sem_ia, sem_ib, sem_oa, sem_ob):
    cid = lax.axis_index("c")
    sid = lax.axis_index("s")
    wid = cid * NS + sid
    m = wid // W_PER_MESH
    c0 = (wid % W_PER_MESH) * ROWS_PER_W

    ibufs = [((ia0, ia1, ia2), sem_ia), ((ib0, ib1, ib2), sem_ib)]
    obufs = [(out_a, sem_oa), (out_b, sem_ob)]

    iota2 = lax.iota(jnp.int32, L) * 2

    def idx_dma(fc):
        ibs, s_i = ibufs[fc % 2]
        return [
            pltpu.async_copy(
                ring_hbm.at[m, k, pl.ds(fc * GC * L, GC * L)], ibs[k], s_i)
            for k in range(K)
        ]

    def row_body(r, carry):
        row = m * C + c0 + r
        pltpu.sync_copy(fea_hbm.at[pl.ds(row * F, F)], fea_buf.at[pl.ds(0, F)])

        h_idx = [None] * NCHUNK
        h_out = [None] * NCHUNK
        h_idx[0] = idx_dma(0)

        for fc in range(NCHUNK):
            ibs = ibufs[fc % 2][0]
            ob, s_o = obufs[fc % 2]
            if fc + 1 < NCHUNK:
                h_idx[fc + 1] = idx_dma(fc + 1)
            for h in h_idx[fc]:
                h.wait()
            if fc >= 2:
                h_out[fc - 2].wait()
            cb = fc * FC

            @plsc.parallel_loop(0, GC, 1, unroll=4)
            def group_body(g, ibs=ibs, ob=ob, cb=cb):
                x0 = ibs[0][pl.ds(g * L, L)]
                x1 = ibs[1][pl.ds(g * L, L)]
                x2 = ibs[2][pl.ds(g * L, L)]
                lo0 = x0 & 0xFFFF
                lo1 = x1 & 0xFFFF
                lo2 = x2 & 0xFFFF
                hi0 = lax.shift_right_logical(x0, 16)
                hi1 = lax.shift_right_logical(x1, 16)
                hi2 = lax.shift_right_logical(x2, 16)
                ev = iota2 + (cb + g * 2 * L)
                od = ev + 1
                v_ev = plsc.load_gather(fea_buf, [ev])
                v_od = plsc.load_gather(fea_buf, [od])
                g0 = plsc.load_gather(fea_buf, [lo0])
                g1 = plsc.load_gather(fea_buf, [lo1])
                g2 = plsc.load_gather(fea_buf, [lo2])
                g3 = plsc.load_gather(fea_buf, [hi0])
                g4 = plsc.load_gather(fea_buf, [hi1])
                g5 = plsc.load_gather(fea_buf, [hi2])
                oev = iota2 + g * 2 * L
                plsc.store_scatter(ob, [oev], jnp.maximum(
                    jnp.maximum(v_ev, g0), jnp.maximum(g1, g2)))
                plsc.store_scatter(ob, [oev + 1], jnp.maximum(
                    jnp.maximum(v_od, g3), jnp.maximum(g4, g5)))

            n_out = FC if fc + 1 < NCHUNK else F_LAST
            h_out[fc] = pltpu.async_copy(
                ob.at[pl.ds(0, n_out)],
                out_hbm.at[pl.ds(row * F + cb, n_out)], s_o)

        h_out[NCHUNK - 2].wait()
        h_out[NCHUNK - 1].wait()
        return carry

    lax.fori_loop(0, ROWS_PER_W, row_body, 0)


_sc_pool = functools.partial(
    pl.kernel,
    mesh=plsc.VectorSubcoreMesh(core_axis_name="c", subcore_axis_name="s"),
    compiler_params=pltpu.CompilerParams(
        needs_layout_passes=False, use_tc_tiling_on_sc=False),
    out_type=jax.ShapeDtypeStruct((M * C * F,), jnp.float32),
    scratch_types=[
        pltpu.VMEM((FP,), jnp.float32),
        pltpu.VMEM((GC * L,), jnp.int32),
        pltpu.VMEM((GC * L,), jnp.int32),
        pltpu.VMEM((GC * L,), jnp.int32),
        pltpu.VMEM((GC * L,), jnp.int32),
        pltpu.VMEM((GC * L,), jnp.int32),
        pltpu.VMEM((GC * L,), jnp.int32),
        pltpu.VMEM((FC,), jnp.float32),
        pltpu.VMEM((FC,), jnp.float32),
        pltpu.SemaphoreType.DMA,
        pltpu.SemaphoreType.DMA,
        pltpu.SemaphoreType.DMA,
        pltpu.SemaphoreType.DMA,
    ],
)(_sc_body)


def kernel(fea, ring_n):
    ring_p = jnp.pad(ring_n, ((0, 0), (0, FP - F), (0, 0)))
    packed = ring_p[:, 0::2, :] | (ring_p[:, 1::2, :] << 16)
    packed = jnp.transpose(packed, (0, 2, 1))
    return _sc_pool(fea.reshape(-1), packed).reshape(M, C, F)
